# Initial kernel scaffold; baseline (speedup 1.0000x reference)
import jax
import jax.numpy as jnp
from jax.experimental import pallas as pl


def kernel(abs_pos, vel_hist, edge_index, tag):
    raise NotImplementedError("write your pallas kernel here")


# speedup vs baseline: 78.9936x; 78.9936x over previous
import jax
import jax.numpy as jnp
from jax import lax
from jax.experimental import pallas as pl
from jax.experimental.pallas import tpu as pltpu
from jax.experimental.pallas import tpu_sc as plsc

N = 50000
DIM = 3
E = 1600000
NC = 2
NS = 16
L = 16
NW = NC * NS
B = 128
KB = 16
C = B * KB
NCHUNK = 25
EW = C * NCHUNK
E2 = EW * NW
N2 = 51200
SLICE = N2 // NS

SIGMA = 3.0 / 359.0 / 3.141592653589793
P_REF = 100.0
ETA_IJ = 2.0 * 0.01 * 0.01 / (0.01 + 0.01 + 1e-08)


def _rsqrt(d2):
    i = plsc.bitcast(d2, jnp.int32)
    y = plsc.bitcast(jnp.int32(0x5F3759DF) - (i >> 1), jnp.float32)
    for _ in range(3):
        y = y * (1.5 - 0.5 * d2 * y * y)
    return y


def _load_indices(ei, ej, base, idx_i2, idx_j2, semi):
    cps = []
    for j in range(KB):
        sl = pl.ds(base + j * B, B)
        cps.append(pltpu.async_copy(ei.at[sl], idx_i2.at[j], semi))
        cps.append(pltpu.async_copy(ej.at[sl], idx_j2.at[j], semi))
    for cp in cps:
        cp.wait()


def _gather_rows(table, idx_i2, idx_j2, rows_i, rows_j, sem1, sem2):
    cps = []
    for j in range(KB):
        sl = pl.ds(j * B, B)
        cps.append(pltpu.async_copy(table.at[idx_i2.at[j]], rows_i.at[sl], sem1))
        cps.append(pltpu.async_copy(table.at[idx_j2.at[j]], rows_j.at[sl], sem2))
    for cp in cps:
        cp.wait()


def _pass1_body(rt, ei, ej, z1, rho_out, dxc, dyc, dzc, gc,
                idx_i2, idx_j2, rows_i, rows_j, wb, dxb, dyb, dzb, gb,
                rho_sh, sem1, sem2, semi):
    cid = lax.axis_index("c")
    sid = lax.axis_index("s")
    wid = sid * NC + cid
    pltpu.sync_copy(z1, rho_sh.at[pl.ds(sid * SLICE, SLICE)])
    plsc.subcore_barrier()
    iota = lax.iota(jnp.int32, L)
    c0 = jnp.full((L,), 0, jnp.int32)
    c1 = jnp.full((L,), 1, jnp.int32)
    c2 = jnp.full((L,), 2, jnp.int32)

    def chunk(k, _):
        base = pl.multiple_of(wid * EW + k * C, 8)
        _load_indices(ei, ej, base, idx_i2, idx_j2, semi)
        _gather_rows(rt, idx_i2, idx_j2, rows_i, rows_j, sem1, sem2)

        def grp(l, _):
            row = l * L + iota
            rix = plsc.load_gather(rows_i, [row, c0])
            riy = plsc.load_gather(rows_i, [row, c1])
            riz = plsc.load_gather(rows_i, [row, c2])
            rjx = plsc.load_gather(rows_j, [row, c0])
            rjy = plsc.load_gather(rows_j, [row, c1])
            rjz = plsc.load_gather(rows_j, [row, c2])
            dx = rix - rjx
            dy = riy - rjy
            dz = riz - rjz
            d2 = dx * dx + dy * dy + dz * dz + 1e-16
            y = _rsqrt(d2)
            d = d2 * y
            q1 = jnp.maximum(0.0, 1.0 - d)
            q2 = jnp.maximum(0.0, 2.0 - d)
            q3 = jnp.maximum(0.0, 3.0 - d)
            q12 = q1 * q1
            q14 = q12 * q12
            q22 = q2 * q2
            q24 = q22 * q22
            q32 = q3 * q3
            q34 = q32 * q32
            w = SIGMA * (q34 * q3 - 6.0 * (q24 * q2) + 15.0 * (q14 * q1))
            gw = (-5.0 * SIGMA) * (q34 - 6.0 * q24 + 15.0 * q14)
            g = gw / (d + 1e-08)
            sl = pl.ds(l * L, L)
            wb[sl] = w
            dxb[sl] = dx
            dyb[sl] = dy
            dzb[sl] = dz
            gb[sl] = g
            return _

        lax.fori_loop(0, C // L, grp, None)
        for j in range(KB):
            pltpu.sync_copy(wb.at[pl.ds(j * B, B)], rho_sh.at[idx_i2.at[j]],
                            add=True)
        pltpu.sync_copy(dxb, dxc.at[pl.ds(base, C)])
        pltpu.sync_copy(dyb, dyc.at[pl.ds(base, C)])
        pltpu.sync_copy(dzb, dzc.at[pl.ds(base, C)])
        pltpu.sync_copy(gb, gc.at[pl.ds(base, C)])
        return _

    lax.fori_loop(0, NCHUNK, chunk, None)
    plsc.subcore_barrier()
    sl = pl.ds(sid * SLICE, SLICE)
    pltpu.sync_copy(rho_sh.at[sl], rho_out.at[cid, sl])


def _pass2_body(vr, ei, ej, dxc, dyc, dzc, gc, z4, dudt_out,
                idx_i2, idx_j2, rows_i, rows_j, dxb, dyb, dzb, gb, abuf,
                dudt_sh, sem1, sem2, semi):
    cid = lax.axis_index("c")
    sid = lax.axis_index("s")
    wid = sid * NC + cid
    pltpu.sync_copy(z4, dudt_sh.at[pl.ds(sid * SLICE, SLICE)])
    pltpu.sync_copy(z4.at[pl.ds(0, C)], abuf)
    plsc.subcore_barrier()
    iota = lax.iota(jnp.int32, L)
    c0 = jnp.full((L,), 0, jnp.int32)
    c1 = jnp.full((L,), 1, jnp.int32)
    c2 = jnp.full((L,), 2, jnp.int32)
    c3 = jnp.full((L,), 3, jnp.int32)

    def chunk(k, _):
        base = pl.multiple_of(wid * EW + k * C, 8)
        _load_indices(ei, ej, base, idx_i2, idx_j2, semi)
        pltpu.sync_copy(dxc.at[pl.ds(base, C)], dxb)
        pltpu.sync_copy(dyc.at[pl.ds(base, C)], dyb)
        pltpu.sync_copy(dzc.at[pl.ds(base, C)], dzb)
        pltpu.sync_copy(gc.at[pl.ds(base, C)], gb)
        _gather_rows(vr, idx_i2, idx_j2, rows_i, rows_j, sem1, sem2)

        def grp(l, _):
            row = l * L + iota
            vix = plsc.load_gather(rows_i, [row, c0])
            viy = plsc.load_gather(rows_i, [row, c1])
            viz = plsc.load_gather(rows_i, [row, c2])
            ri = plsc.load_gather(rows_i, [row, c3])
            vjx = plsc.load_gather(rows_j, [row, c0])
            vjy = plsc.load_gather(rows_j, [row, c1])
            vjz = plsc.load_gather(rows_j, [row, c2])
            rj = plsc.load_gather(rows_j, [row, c3])
            sl = pl.ds(l * L, L)
            dx = dxb[sl]
            dy = dyb[sl]
            dz = dzb[sl]
            g = gb[sl]
            inv_i = 1.0 / ri
            inv_j = 1.0 / rj
            cc = (inv_i * inv_i + inv_j * inv_j) * g
            num = P_REF * (2.0 * ri * rj - ri - rj)
            p_ij = num / (ri + rj)
            ax = cc * (-p_ij * dx + ETA_IJ * (vix - vjx))
            ay = cc * (-p_ij * dy + ETA_IJ * (viy - vjy))
            az = cc * (-p_ij * dz + ETA_IJ * (viz - vjz))
            plsc.store_scatter(abuf, [row, c0], ax)
            plsc.store_scatter(abuf, [row, c1], ay)
            plsc.store_scatter(abuf, [row, c2], az)
            return _

        lax.fori_loop(0, C // L, grp, None)
        for j in range(KB):
            pltpu.sync_copy(abuf.at[pl.ds(j * B, B)], dudt_sh.at[idx_i2.at[j]],
                            add=True)
        return _

    lax.fori_loop(0, NCHUNK, chunk, None)
    plsc.subcore_barrier()
    sl = pl.ds(sid * SLICE, SLICE)
    pltpu.sync_copy(dudt_sh.at[sl], dudt_out.at[cid, sl])


def _sum_rho_body(part_ref, rho_ref, p_ref):
    s = part_ref[0] + part_ref[1]
    rho_ref[...] = s
    p_ref[...] = P_REF * (s - 1.0)


def _sum_dudt_body(part_ref, out_ref):
    out_ref[...] = part_ref[0] + part_ref[1]


@jax.jit
def kernel(abs_pos, vel_hist, edge_index, tag):
    del tag
    f32 = jnp.float32
    r = abs_pos[..., -1].astype(f32)
    rt = jnp.pad(r, ((0, N2 - N), (0, 5)))
    ei = jnp.pad(edge_index[0].astype(jnp.int32), (0, E2 - E),
                 constant_values=N)
    ej = jnp.pad(edge_index[1].astype(jnp.int32), (0, E2 - E),
                 constant_values=N)
    z1 = jnp.zeros((SLICE,), f32)
    z4 = jnp.zeros((SLICE, 8), f32)

    mesh = plsc.VectorSubcoreMesh(
        core_axis_name="c", subcore_axis_name="s",
        num_cores=NC, num_subcores=NS)
    cparams = pltpu.CompilerParams(
        use_tc_tiling_on_sc=False, needs_layout_passes=False)

    pass1 = pl.kernel(
        _pass1_body,
        out_type=[
            jax.ShapeDtypeStruct((NC, N2), f32),
            jax.ShapeDtypeStruct((E2,), f32),
            jax.ShapeDtypeStruct((E2,), f32),
            jax.ShapeDtypeStruct((E2,), f32),
            jax.ShapeDtypeStruct((E2,), f32),
        ],
        mesh=mesh,
        scratch_types=[
            pltpu.VMEM((KB, B), jnp.int32),
            pltpu.VMEM((KB, B), jnp.int32),
            pltpu.VMEM((C, 8), f32),
            pltpu.VMEM((C, 8), f32),
            pltpu.VMEM((C,), f32),
            pltpu.VMEM((C,), f32),
            pltpu.VMEM((C,), f32),
            pltpu.VMEM((C,), f32),
            pltpu.VMEM((C,), f32),
            pltpu.VMEM_SHARED((N2,), f32),
            pltpu.SemaphoreType.DMA,
            pltpu.SemaphoreType.DMA,
            pltpu.SemaphoreType.DMA,
        ],
        compiler_params=cparams,
    )
    rho_part, dxc, dyc, dzc, gc = pass1(rt, ei, ej, z1)

    rho2d, p2d = pl.pallas_call(
        _sum_rho_body,
        out_shape=[
            jax.ShapeDtypeStruct((N2 // 128, 128), f32),
            jax.ShapeDtypeStruct((N2 // 128, 128), f32),
        ],
    )(rho_part.reshape(NC, N2 // 128, 128))
    rho_full = rho2d.reshape(-1)
    rho = rho_full[:N]
    p = p2d.reshape(-1)[:N]

    vr = jnp.concatenate(
        [jnp.pad(vel_hist.astype(f32), ((0, N2 - N), (0, 0))),
         rho_full[:, None],
         jnp.zeros((N2, 4), f32)], axis=1)

    pass2 = pl.kernel(
        _pass2_body,
        out_type=jax.ShapeDtypeStruct((NC, N2, 8), f32),
        mesh=mesh,
        scratch_types=[
            pltpu.VMEM((KB, B), jnp.int32),
            pltpu.VMEM((KB, B), jnp.int32),
            pltpu.VMEM((C, 8), f32),
            pltpu.VMEM((C, 8), f32),
            pltpu.VMEM((C,), f32),
            pltpu.VMEM((C,), f32),
            pltpu.VMEM((C,), f32),
            pltpu.VMEM((C,), f32),
            pltpu.VMEM((C, 8), f32),
            pltpu.VMEM_SHARED((N2, 8), f32),
            pltpu.SemaphoreType.DMA,
            pltpu.SemaphoreType.DMA,
            pltpu.SemaphoreType.DMA,
        ],
        compiler_params=cparams,
    )
    dudt_part = pass2(vr, ei, ej, dxc, dyc, dzc, gc, z4)

    dudt4 = pl.pallas_call(
        _sum_dudt_body,
        out_shape=jax.ShapeDtypeStruct((N2 * 8 // 128, 128), f32),
    )(dudt_part.reshape(NC, N2 * 8 // 128, 128))
    dudt = dudt4.reshape(N2, 8)[:N, :3]

    dvdt = jnp.zeros((N, DIM), f32)
    return dudt, dvdt, rho, p


# speedup vs baseline: 79.6384x; 1.0082x over previous
import jax
import jax.numpy as jnp
from jax import lax
from jax.experimental import pallas as pl
from jax.experimental.pallas import tpu as pltpu
from jax.experimental.pallas import tpu_sc as plsc

N = 50000
DIM = 3
E = 1600000
NC = 2
NS = 16
L = 16
NW = NC * NS
B = 128
KB = 16
C = B * KB
NCHUNK = 25
EW = C * NCHUNK
E2 = EW * NW
N2 = 51200
SLICE = N2 // NS

SIGMA = 3.0 / 359.0 / 3.141592653589793
P_REF = 100.0
ETA_IJ = 2.0 * 0.01 * 0.01 / (0.01 + 0.01 + 1e-08)


def _rsqrt(d2):
    i = plsc.bitcast(d2, jnp.int32)
    y = plsc.bitcast(jnp.int32(0x5F3759DF) - (i >> 1), jnp.float32)
    for _ in range(3):
        y = y * (1.5 - 0.5 * d2 * y * y)
    return y


def _load_indices(ei, ej, base, idx_i, idx_j, semi):
    cp1 = pltpu.async_copy(ei.at[pl.ds(base, C)], idx_i, semi)
    cp2 = pltpu.async_copy(ej.at[pl.ds(base, C)], idx_j, semi)
    cp1.wait()
    cp2.wait()


def _gather_rows(table, idx_i, idx_j, rows_i, rows_j, sem1, sem2):
    cp1 = pltpu.async_copy(table.at[idx_i], rows_i, sem1)
    cp2 = pltpu.async_copy(table.at[idx_j], rows_j, sem2)
    cp1.wait()
    cp2.wait()


def _pass1_body(rt, ei, ej, z1, rho_out, dxc, dyc, dzc, gc,
                idx_i, idx_j, rows_i, rows_j, wb, dxb, dyb, dzb, gb,
                rho_sh, sem1, sem2, semi):
    cid = lax.axis_index("c")
    sid = lax.axis_index("s")
    wid = sid * NC + cid
    pltpu.sync_copy(z1, rho_sh.at[pl.ds(sid * SLICE, SLICE)])
    plsc.subcore_barrier()
    iota = lax.iota(jnp.int32, L)
    c0 = jnp.full((L,), 0, jnp.int32)
    c1 = jnp.full((L,), 1, jnp.int32)
    c2 = jnp.full((L,), 2, jnp.int32)

    def chunk(k, _):
        base = pl.multiple_of(wid * EW + k * C, 8)
        _load_indices(ei, ej, base, idx_i, idx_j, semi)
        _gather_rows(rt, idx_i, idx_j, rows_i, rows_j, sem1, sem2)

        def grp(l, _):
            row = l * L + iota
            rix = plsc.load_gather(rows_i, [row, c0])
            riy = plsc.load_gather(rows_i, [row, c1])
            riz = plsc.load_gather(rows_i, [row, c2])
            rjx = plsc.load_gather(rows_j, [row, c0])
            rjy = plsc.load_gather(rows_j, [row, c1])
            rjz = plsc.load_gather(rows_j, [row, c2])
            dx = rix - rjx
            dy = riy - rjy
            dz = riz - rjz
            d2 = dx * dx + dy * dy + dz * dz + 1e-16
            y = _rsqrt(d2)
            d = d2 * y
            q1 = jnp.maximum(0.0, 1.0 - d)
            q2 = jnp.maximum(0.0, 2.0 - d)
            q3 = jnp.maximum(0.0, 3.0 - d)
            q12 = q1 * q1
            q14 = q12 * q12
            q22 = q2 * q2
            q24 = q22 * q22
            q32 = q3 * q3
            q34 = q32 * q32
            w = SIGMA * (q34 * q3 - 6.0 * (q24 * q2) + 15.0 * (q14 * q1))
            gw = (-5.0 * SIGMA) * (q34 - 6.0 * q24 + 15.0 * q14)
            g = gw / (d + 1e-08)
            sl = pl.ds(l * L, L)
            wb[sl] = w
            dxb[sl] = dx
            dyb[sl] = dy
            dzb[sl] = dz
            gb[sl] = g
            return _

        lax.fori_loop(0, C // L, grp, None)
        pltpu.sync_copy(wb, rho_sh.at[idx_i], add=True)
        pltpu.sync_copy(dxb, dxc.at[pl.ds(base, C)])
        pltpu.sync_copy(dyb, dyc.at[pl.ds(base, C)])
        pltpu.sync_copy(dzb, dzc.at[pl.ds(base, C)])
        pltpu.sync_copy(gb, gc.at[pl.ds(base, C)])
        return _

    lax.fori_loop(0, NCHUNK, chunk, None)
    plsc.subcore_barrier()
    sl = pl.ds(sid * SLICE, SLICE)
    pltpu.sync_copy(rho_sh.at[sl], rho_out.at[cid, sl])


def _pass2_body(vr, ei, ej, dxc, dyc, dzc, gc, z4, dudt_out,
                idx_i, idx_j, rows_i, rows_j, dxb, dyb, dzb, gb, abuf,
                dudt_sh, sem1, sem2, semi):
    cid = lax.axis_index("c")
    sid = lax.axis_index("s")
    wid = sid * NC + cid
    pltpu.sync_copy(z4, dudt_sh.at[pl.ds(sid * SLICE, SLICE)])
    pltpu.sync_copy(z4.at[pl.ds(0, C)], abuf)
    plsc.subcore_barrier()
    iota = lax.iota(jnp.int32, L)
    c0 = jnp.full((L,), 0, jnp.int32)
    c1 = jnp.full((L,), 1, jnp.int32)
    c2 = jnp.full((L,), 2, jnp.int32)
    c3 = jnp.full((L,), 3, jnp.int32)

    def chunk(k, _):
        base = pl.multiple_of(wid * EW + k * C, 8)
        _load_indices(ei, ej, base, idx_i, idx_j, semi)
        pltpu.sync_copy(dxc.at[pl.ds(base, C)], dxb)
        pltpu.sync_copy(dyc.at[pl.ds(base, C)], dyb)
        pltpu.sync_copy(dzc.at[pl.ds(base, C)], dzb)
        pltpu.sync_copy(gc.at[pl.ds(base, C)], gb)
        _gather_rows(vr, idx_i, idx_j, rows_i, rows_j, sem1, sem2)

        def grp(l, _):
            row = l * L + iota
            vix = plsc.load_gather(rows_i, [row, c0])
            viy = plsc.load_gather(rows_i, [row, c1])
            viz = plsc.load_gather(rows_i, [row, c2])
            ri = plsc.load_gather(rows_i, [row, c3])
            vjx = plsc.load_gather(rows_j, [row, c0])
            vjy = plsc.load_gather(rows_j, [row, c1])
            vjz = plsc.load_gather(rows_j, [row, c2])
            rj = plsc.load_gather(rows_j, [row, c3])
            sl = pl.ds(l * L, L)
            dx = dxb[sl]
            dy = dyb[sl]
            dz = dzb[sl]
            g = gb[sl]
            inv_i = 1.0 / ri
            inv_j = 1.0 / rj
            cc = (inv_i * inv_i + inv_j * inv_j) * g
            num = P_REF * (2.0 * ri * rj - ri - rj)
            p_ij = num / (ri + rj)
            ax = cc * (-p_ij * dx + ETA_IJ * (vix - vjx))
            ay = cc * (-p_ij * dy + ETA_IJ * (viy - vjy))
            az = cc * (-p_ij * dz + ETA_IJ * (viz - vjz))
            plsc.store_scatter(abuf, [row, c0], ax)
            plsc.store_scatter(abuf, [row, c1], ay)
            plsc.store_scatter(abuf, [row, c2], az)
            return _

        lax.fori_loop(0, C // L, grp, None)
        pltpu.sync_copy(abuf, dudt_sh.at[idx_i], add=True)
        return _

    lax.fori_loop(0, NCHUNK, chunk, None)
    plsc.subcore_barrier()
    sl = pl.ds(sid * SLICE, SLICE)
    pltpu.sync_copy(dudt_sh.at[sl], dudt_out.at[cid, sl])


def _sum_rho_body(part_ref, rho_ref, p_ref):
    s = part_ref[0] + part_ref[1]
    rho_ref[...] = s
    p_ref[...] = P_REF * (s - 1.0)


def _sum_dudt_body(part_ref, out_ref):
    out_ref[...] = part_ref[0] + part_ref[1]


@jax.jit
def kernel(abs_pos, vel_hist, edge_index, tag):
    del tag
    f32 = jnp.float32
    r = abs_pos[..., -1].astype(f32)
    rt = jnp.pad(r, ((0, N2 - N), (0, 5)))
    ei = jnp.pad(edge_index[0].astype(jnp.int32), (0, E2 - E),
                 constant_values=N)
    ej = jnp.pad(edge_index[1].astype(jnp.int32), (0, E2 - E),
                 constant_values=N)
    z1 = jnp.zeros((SLICE,), f32)
    z4 = jnp.zeros((SLICE, 8), f32)

    mesh = plsc.VectorSubcoreMesh(
        core_axis_name="c", subcore_axis_name="s",
        num_cores=NC, num_subcores=NS)
    cparams = pltpu.CompilerParams(
        use_tc_tiling_on_sc=False, needs_layout_passes=False)

    pass1 = pl.kernel(
        _pass1_body,
        out_type=[
            jax.ShapeDtypeStruct((NC, N2), f32),
            jax.ShapeDtypeStruct((E2,), f32),
            jax.ShapeDtypeStruct((E2,), f32),
            jax.ShapeDtypeStruct((E2,), f32),
            jax.ShapeDtypeStruct((E2,), f32),
        ],
        mesh=mesh,
        scratch_types=[
            pltpu.VMEM((C,), jnp.int32),
            pltpu.VMEM((C,), jnp.int32),
            pltpu.VMEM((C, 8), f32),
            pltpu.VMEM((C, 8), f32),
            pltpu.VMEM((C,), f32),
            pltpu.VMEM((C,), f32),
            pltpu.VMEM((C,), f32),
            pltpu.VMEM((C,), f32),
            pltpu.VMEM((C,), f32),
            pltpu.VMEM_SHARED((N2,), f32),
            pltpu.SemaphoreType.DMA,
            pltpu.SemaphoreType.DMA,
            pltpu.SemaphoreType.DMA,
        ],
        compiler_params=cparams,
    )
    rho_part, dxc, dyc, dzc, gc = pass1(rt, ei, ej, z1)

    rho2d, p2d = pl.pallas_call(
        _sum_rho_body,
        out_shape=[
            jax.ShapeDtypeStruct((N2 // 128, 128), f32),
            jax.ShapeDtypeStruct((N2 // 128, 128), f32),
        ],
    )(rho_part.reshape(NC, N2 // 128, 128))
    rho_full = rho2d.reshape(-1)
    rho = rho_full[:N]
    p = p2d.reshape(-1)[:N]

    vr = jnp.concatenate(
        [jnp.pad(vel_hist.astype(f32), ((0, N2 - N), (0, 0))),
         rho_full[:, None],
         jnp.zeros((N2, 4), f32)], axis=1)

    pass2 = pl.kernel(
        _pass2_body,
        out_type=jax.ShapeDtypeStruct((NC, N2, 8), f32),
        mesh=mesh,
        scratch_types=[
            pltpu.VMEM((C,), jnp.int32),
            pltpu.VMEM((C,), jnp.int32),
            pltpu.VMEM((C, 8), f32),
            pltpu.VMEM((C, 8), f32),
            pltpu.VMEM((C,), f32),
            pltpu.VMEM((C,), f32),
            pltpu.VMEM((C,), f32),
            pltpu.VMEM((C,), f32),
            pltpu.VMEM((C, 8), f32),
            pltpu.VMEM_SHARED((N2, 8), f32),
            pltpu.SemaphoreType.DMA,
            pltpu.SemaphoreType.DMA,
            pltpu.SemaphoreType.DMA,
        ],
        compiler_params=cparams,
    )
    dudt_part = pass2(vr, ei, ej, dxc, dyc, dzc, gc, z4)

    dudt4 = pl.pallas_call(
        _sum_dudt_body,
        out_shape=jax.ShapeDtypeStruct((N2 * 8 // 128, 128), f32),
    )(dudt_part.reshape(NC, N2 * 8 // 128, 128))
    dudt = dudt4.reshape(N2, 8)[:N, :3]

    dvdt = jnp.zeros((N, DIM), f32)
    return dudt, dvdt, rho, p


# speedup vs baseline: 175.5513x; 2.2044x over previous
import jax
import jax.numpy as jnp
from jax import lax
from jax.experimental import pallas as pl
from jax.experimental.pallas import tpu as pltpu
from jax.experimental.pallas import tpu_sc as plsc

N = 50000
DIM = 3
E = 1600000
NC = 2
NS = 16
L = 16
NW = NC * NS
EW = E // NW
C = 2000
NCHUNK = EW // C
N2 = 51200
SLICE = N2 // NS

SIGMA = 3.0 / 359.0 / 3.141592653589793
P_REF = 100.0
ETA_IJ = 2.0 * 0.01 * 0.01 / (0.01 + 0.01 + 1e-08)


def _rsqrt(d2):
    i = plsc.bitcast(d2, jnp.int32)
    y = plsc.bitcast(jnp.int32(0x5F3759DF) - (i >> 1), jnp.float32)
    for _ in range(3):
        y = y * (1.5 - 0.5 * d2 * y * y)
    return y


def _slot(ref, s):
    return ref.at[pl.ds(s * C, C)]


def _fetch(table, ei, ej, base, idx_i, idx_j, rows_i, rows_j, s, semi, semj):
    pltpu.sync_copy(ei.at[pl.ds(base, C)], _slot(idx_i, s))
    pltpu.sync_copy(ej.at[pl.ds(base, C)], _slot(idx_j, s))
    pltpu.async_copy(table.at[_slot(idx_i, s)], _slot(rows_i, s), semi)
    pltpu.async_copy(table.at[_slot(idx_j, s)], _slot(rows_j, s), semj)


def _drain(table, idx_i, idx_j, rows_i, rows_j, s, semi, semj):
    pltpu.make_async_copy(table.at[_slot(idx_i, s)], _slot(rows_i, s),
                          semi).wait()
    pltpu.make_async_copy(table.at[_slot(idx_j, s)], _slot(rows_j, s),
                          semj).wait()


def _pass1_body(rt, ei, ej, z1, rho_out, dxc, dyc, dzc, gc,
                idx_i, idx_j, rows_i, rows_j, wb, dxb, dyb, dzb, gb,
                rho_sh, sA1, sA2, sB1, sB2):
    cid = lax.axis_index("c")
    sid = lax.axis_index("s")
    wid = sid * NC + cid
    e0 = wid * EW
    pltpu.sync_copy(z1, rho_sh.at[pl.ds(sid * SLICE, SLICE)])
    plsc.subcore_barrier()
    iota = lax.iota(jnp.int32, L)
    c0 = jnp.full((L,), 0, jnp.int32)
    c1 = jnp.full((L,), 1, jnp.int32)
    c2 = jnp.full((L,), 2, jnp.int32)

    def compute(k, s):
        rows_i_s = _slot(rows_i, s)
        rows_j_s = _slot(rows_j, s)

        def grp(l, _):
            row = l * L + iota
            rix = plsc.load_gather(rows_i_s, [row, c0])
            riy = plsc.load_gather(rows_i_s, [row, c1])
            riz = plsc.load_gather(rows_i_s, [row, c2])
            rjx = plsc.load_gather(rows_j_s, [row, c0])
            rjy = plsc.load_gather(rows_j_s, [row, c1])
            rjz = plsc.load_gather(rows_j_s, [row, c2])
            dx = rix - rjx
            dy = riy - rjy
            dz = riz - rjz
            d2 = dx * dx + dy * dy + dz * dz + 1e-16
            y = _rsqrt(d2)
            d = d2 * y
            q1 = jnp.maximum(0.0, 1.0 - d)
            q2 = jnp.maximum(0.0, 2.0 - d)
            q3 = jnp.maximum(0.0, 3.0 - d)
            q12 = q1 * q1
            q14 = q12 * q12
            q22 = q2 * q2
            q24 = q22 * q22
            q32 = q3 * q3
            q34 = q32 * q32
            w = SIGMA * (q34 * q3 - 6.0 * (q24 * q2) + 15.0 * (q14 * q1))
            gw = (-5.0 * SIGMA) * (q34 - 6.0 * q24 + 15.0 * q14)
            g = gw / (d + 1e-08)
            sl = pl.ds(l * L, L)
            wb[sl] = w
            dxb[sl] = dx
            dyb[sl] = dy
            dzb[sl] = dz
            gb[sl] = g
            return _

        lax.fori_loop(0, C // L, grp, None)
        base = pl.multiple_of(e0 + k * C, 8)
        pltpu.sync_copy(wb, rho_sh.at[_slot(idx_i, s)], add=True)
        pltpu.sync_copy(dxb, dxc.at[pl.ds(base, C)])
        pltpu.sync_copy(dyb, dyc.at[pl.ds(base, C)])
        pltpu.sync_copy(dzb, dzc.at[pl.ds(base, C)])
        pltpu.sync_copy(gb, gc.at[pl.ds(base, C)])

    _fetch(rt, ei, ej, e0, idx_i, idx_j, rows_i, rows_j, 0, sA1, sA2)

    def two_chunks(m, _):
        k0 = 2 * m
        _fetch(rt, ei, ej, e0 + (k0 + 1) * C, idx_i, idx_j, rows_i, rows_j,
               1, sB1, sB2)
        _drain(rt, idx_i, idx_j, rows_i, rows_j, 0, sA1, sA2)
        compute(k0, 0)

        @pl.when(k0 + 2 < NCHUNK)
        def _prefetch():
            _fetch(rt, ei, ej, e0 + (k0 + 2) * C, idx_i, idx_j, rows_i,
                   rows_j, 0, sA1, sA2)

        _drain(rt, idx_i, idx_j, rows_i, rows_j, 1, sB1, sB2)
        compute(k0 + 1, 1)
        return _

    lax.fori_loop(0, NCHUNK // 2, two_chunks, None)
    _drain(rt, idx_i, idx_j, rows_i, rows_j, 0, sA1, sA2)
    compute(NCHUNK - 1, 0)

    plsc.subcore_barrier()
    sl = pl.ds(sid * SLICE, SLICE)
    pltpu.sync_copy(rho_sh.at[sl], rho_out.at[cid, sl])


def _pass2_body(vr, ei, ej, dxc, dyc, dzc, gc, z8, dudt_out,
                idx_i, idx_j, rows_i, rows_j, dxb, dyb, dzb, gb, abuf,
                dudt_sh, sA1, sA2, sB1, sB2):
    cid = lax.axis_index("c")
    sid = lax.axis_index("s")
    wid = sid * NC + cid
    e0 = wid * EW
    pltpu.sync_copy(z8, dudt_sh.at[pl.ds(sid * SLICE, SLICE)])
    pltpu.sync_copy(z8.at[pl.ds(0, C)], abuf)
    plsc.subcore_barrier()
    iota = lax.iota(jnp.int32, L)
    c0 = jnp.full((L,), 0, jnp.int32)
    c1 = jnp.full((L,), 1, jnp.int32)
    c2 = jnp.full((L,), 2, jnp.int32)
    c3 = jnp.full((L,), 3, jnp.int32)

    def compute(k, s):
        rows_i_s = _slot(rows_i, s)
        rows_j_s = _slot(rows_j, s)
        base = pl.multiple_of(e0 + k * C, 8)
        pltpu.sync_copy(dxc.at[pl.ds(base, C)], dxb)
        pltpu.sync_copy(dyc.at[pl.ds(base, C)], dyb)
        pltpu.sync_copy(dzc.at[pl.ds(base, C)], dzb)
        pltpu.sync_copy(gc.at[pl.ds(base, C)], gb)

        def grp(l, _):
            row = l * L + iota
            vix = plsc.load_gather(rows_i_s, [row, c0])
            viy = plsc.load_gather(rows_i_s, [row, c1])
            viz = plsc.load_gather(rows_i_s, [row, c2])
            ri = plsc.load_gather(rows_i_s, [row, c3])
            vjx = plsc.load_gather(rows_j_s, [row, c0])
            vjy = plsc.load_gather(rows_j_s, [row, c1])
            vjz = plsc.load_gather(rows_j_s, [row, c2])
            rj = plsc.load_gather(rows_j_s, [row, c3])
            sl = pl.ds(l * L, L)
            dx = dxb[sl]
            dy = dyb[sl]
            dz = dzb[sl]
            g = gb[sl]
            inv_i = 1.0 / ri
            inv_j = 1.0 / rj
            cc = (inv_i * inv_i + inv_j * inv_j) * g
            num = P_REF * (2.0 * ri * rj - ri - rj)
            p_ij = num / (ri + rj)
            ax = cc * (-p_ij * dx + ETA_IJ * (vix - vjx))
            ay = cc * (-p_ij * dy + ETA_IJ * (viy - vjy))
            az = cc * (-p_ij * dz + ETA_IJ * (viz - vjz))
            plsc.store_scatter(abuf, [row, c0], ax)
            plsc.store_scatter(abuf, [row, c1], ay)
            plsc.store_scatter(abuf, [row, c2], az)
            return _

        lax.fori_loop(0, C // L, grp, None)
        pltpu.sync_copy(abuf, dudt_sh.at[_slot(idx_i, s)], add=True)

    _fetch(vr, ei, ej, e0, idx_i, idx_j, rows_i, rows_j, 0, sA1, sA2)

    def two_chunks(m, _):
        k0 = 2 * m
        _fetch(vr, ei, ej, e0 + (k0 + 1) * C, idx_i, idx_j, rows_i, rows_j,
               1, sB1, sB2)
        _drain(vr, idx_i, idx_j, rows_i, rows_j, 0, sA1, sA2)
        compute(k0, 0)

        @pl.when(k0 + 2 < NCHUNK)
        def _prefetch():
            _fetch(vr, ei, ej, e0 + (k0 + 2) * C, idx_i, idx_j, rows_i,
                   rows_j, 0, sA1, sA2)

        _drain(vr, idx_i, idx_j, rows_i, rows_j, 1, sB1, sB2)
        compute(k0 + 1, 1)
        return _

    lax.fori_loop(0, NCHUNK // 2, two_chunks, None)
    _drain(vr, idx_i, idx_j, rows_i, rows_j, 0, sA1, sA2)
    compute(NCHUNK - 1, 0)

    plsc.subcore_barrier()
    sl = pl.ds(sid * SLICE, SLICE)
    pltpu.sync_copy(dudt_sh.at[sl], dudt_out.at[cid, sl])


def _sum_rho_body(part_ref, rho_ref, p_ref):
    s = part_ref[0] + part_ref[1]
    rho_ref[...] = s
    p_ref[...] = P_REF * (s - 1.0)


def _sum_dudt_body(part_ref, out_ref):
    out_ref[...] = part_ref[0] + part_ref[1]


@jax.jit
def kernel(abs_pos, vel_hist, edge_index, tag):
    del tag
    f32 = jnp.float32
    r = abs_pos[..., -1].astype(f32)
    rt = jnp.pad(r, ((0, 0), (0, 5)))
    ei = edge_index[0].astype(jnp.int32)
    ej = edge_index[1].astype(jnp.int32)
    z1 = jnp.zeros((SLICE,), f32)
    z8 = jnp.zeros((SLICE, 8), f32)

    mesh = plsc.VectorSubcoreMesh(
        core_axis_name="c", subcore_axis_name="s",
        num_cores=NC, num_subcores=NS)
    cparams = pltpu.CompilerParams(
        use_tc_tiling_on_sc=False, needs_layout_passes=False)

    pass1 = pl.kernel(
        _pass1_body,
        out_type=[
            jax.ShapeDtypeStruct((NC, N2), f32),
            jax.ShapeDtypeStruct((E,), f32),
            jax.ShapeDtypeStruct((E,), f32),
            jax.ShapeDtypeStruct((E,), f32),
            jax.ShapeDtypeStruct((E,), f32),
        ],
        mesh=mesh,
        scratch_types=[
            pltpu.VMEM((2 * C,), jnp.int32),
            pltpu.VMEM((2 * C,), jnp.int32),
            pltpu.VMEM((2 * C, 8), f32),
            pltpu.VMEM((2 * C, 8), f32),
            pltpu.VMEM((C,), f32),
            pltpu.VMEM((C,), f32),
            pltpu.VMEM((C,), f32),
            pltpu.VMEM((C,), f32),
            pltpu.VMEM((C,), f32),
            pltpu.VMEM_SHARED((N2,), f32),
            pltpu.SemaphoreType.DMA,
            pltpu.SemaphoreType.DMA,
            pltpu.SemaphoreType.DMA,
            pltpu.SemaphoreType.DMA,
        ],
        compiler_params=cparams,
    )
    rho_part, dxc, dyc, dzc, gc = pass1(rt, ei, ej, z1)

    rho2d, p2d = pl.pallas_call(
        _sum_rho_body,
        out_shape=[
            jax.ShapeDtypeStruct((N2 // 128, 128), f32),
            jax.ShapeDtypeStruct((N2 // 128, 128), f32),
        ],
    )(rho_part.reshape(NC, N2 // 128, 128))
    rho = rho2d.reshape(-1)[:N]
    p = p2d.reshape(-1)[:N]

    vr = jnp.concatenate(
        [vel_hist.astype(f32), rho[:, None], jnp.zeros((N, 4), f32)],
        axis=1)

    pass2 = pl.kernel(
        _pass2_body,
        out_type=jax.ShapeDtypeStruct((NC, N2, 8), f32),
        mesh=mesh,
        scratch_types=[
            pltpu.VMEM((2 * C,), jnp.int32),
            pltpu.VMEM((2 * C,), jnp.int32),
            pltpu.VMEM((2 * C, 8), f32),
            pltpu.VMEM((2 * C, 8), f32),
            pltpu.VMEM((C,), f32),
            pltpu.VMEM((C,), f32),
            pltpu.VMEM((C,), f32),
            pltpu.VMEM((C,), f32),
            pltpu.VMEM((C, 8), f32),
            pltpu.VMEM_SHARED((N2, 8), f32),
            pltpu.SemaphoreType.DMA,
            pltpu.SemaphoreType.DMA,
            pltpu.SemaphoreType.DMA,
            pltpu.SemaphoreType.DMA,
        ],
        compiler_params=cparams,
    )
    dudt_part = pass2(vr, ei, ej, dxc, dyc, dzc, gc, z8)

    dudt8 = pl.pallas_call(
        _sum_dudt_body,
        out_shape=jax.ShapeDtypeStruct((N2 * 8 // 128, 128), f32),
    )(dudt_part.reshape(NC, N2 * 8 // 128, 128))
    dudt = dudt8.reshape(N2, 8)[:N, :3]

    dvdt = jnp.zeros((N, DIM), f32)
    return dudt, dvdt, rho, p


# speedup vs baseline: 183.0372x; 1.0426x over previous
import jax
import jax.numpy as jnp
from jax import lax
from jax.experimental import pallas as pl
from jax.experimental.pallas import tpu as pltpu
from jax.experimental.pallas import tpu_sc as plsc

N = 50000
DIM = 3
E = 1600000
NC = 2
NS = 16
L = 16
NW = NC * NS
EW = E // NW
C = 2000
NCHUNK = EW // C
N2 = 51200
SLICE = N2 // NS

SIGMA = 3.0 / 359.0 / 3.141592653589793
P_REF = 100.0
ETA_IJ = 2.0 * 0.01 * 0.01 / (0.01 + 0.01 + 1e-08)


def _rsqrt(d2):
    i = plsc.bitcast(d2, jnp.int32)
    y = plsc.bitcast(jnp.int32(0x5F3759DF) - (i >> 1), jnp.float32)
    for _ in range(3):
        y = y * (1.5 - 0.5 * d2 * y * y)
    return y


def _slot(ref, s):
    return ref.at[pl.ds(s * C, C)]


def _fetch(table, ei, ej, base, idx_i, idx_j, rows_i, rows_j, s, semi, semj):
    pltpu.sync_copy(ei.at[pl.ds(base, C)], _slot(idx_i, s))
    pltpu.sync_copy(ej.at[pl.ds(base, C)], _slot(idx_j, s))
    pltpu.async_copy(table.at[_slot(idx_i, s)], _slot(rows_i, s), semi)
    pltpu.async_copy(table.at[_slot(idx_j, s)], _slot(rows_j, s), semj)


def _drain(table, idx_i, idx_j, rows_i, rows_j, s, semi, semj):
    pltpu.make_async_copy(table.at[_slot(idx_i, s)], _slot(rows_i, s),
                          semi).wait()
    pltpu.make_async_copy(table.at[_slot(idx_j, s)], _slot(rows_j, s),
                          semj).wait()


def _pass1_body(rt, ei, ej, z1, rho_out, dxc, dyc, dzc, gc,
                idx_i, idx_j, rows_i, rows_j, wb, dxb, dyb, dzb, gb,
                rho_sh, sA1, sA2, sB1, sB2):
    cid = lax.axis_index("c")
    sid = lax.axis_index("s")
    wid = sid * NC + cid
    e0 = wid * EW
    pltpu.sync_copy(z1, rho_sh.at[pl.ds(sid * SLICE, SLICE)])
    plsc.subcore_barrier()
    iota = lax.iota(jnp.int32, L)
    c0 = jnp.full((L,), 0, jnp.int32)
    c1 = jnp.full((L,), 1, jnp.int32)
    c2 = jnp.full((L,), 2, jnp.int32)

    def compute(k, s):
        rows_i_s = _slot(rows_i, s)
        rows_j_s = _slot(rows_j, s)

        def grp(l, _):
            row = l * L + iota
            rix = plsc.load_gather(rows_i_s, [row, c0])
            riy = plsc.load_gather(rows_i_s, [row, c1])
            riz = plsc.load_gather(rows_i_s, [row, c2])
            rjx = plsc.load_gather(rows_j_s, [row, c0])
            rjy = plsc.load_gather(rows_j_s, [row, c1])
            rjz = plsc.load_gather(rows_j_s, [row, c2])
            dx = rix - rjx
            dy = riy - rjy
            dz = riz - rjz
            d2 = dx * dx + dy * dy + dz * dz + 1e-16
            y = _rsqrt(d2)
            d = d2 * y
            q1 = jnp.maximum(0.0, 1.0 - d)
            q2 = jnp.maximum(0.0, 2.0 - d)
            q3 = jnp.maximum(0.0, 3.0 - d)
            q12 = q1 * q1
            q14 = q12 * q12
            q22 = q2 * q2
            q24 = q22 * q22
            q32 = q3 * q3
            q34 = q32 * q32
            w = SIGMA * (q34 * q3 - 6.0 * (q24 * q2) + 15.0 * (q14 * q1))
            gw = (-5.0 * SIGMA) * (q34 - 6.0 * q24 + 15.0 * q14)
            g = gw / (d + 1e-08)
            sl = pl.ds(l * L, L)
            wb[sl] = w
            dxb[sl] = dx
            dyb[sl] = dy
            dzb[sl] = dz
            gb[sl] = g
            return _

        lax.fori_loop(0, C // L, grp, None)
        base = pl.multiple_of(e0 + k * C, 8)
        pltpu.sync_copy(wb, rho_sh.at[_slot(idx_i, s)], add=True)
        pltpu.sync_copy(dxb, dxc.at[pl.ds(base, C)])
        pltpu.sync_copy(dyb, dyc.at[pl.ds(base, C)])
        pltpu.sync_copy(dzb, dzc.at[pl.ds(base, C)])
        pltpu.sync_copy(gb, gc.at[pl.ds(base, C)])

    _fetch(rt, ei, ej, e0, idx_i, idx_j, rows_i, rows_j, 0, sA1, sA2)

    def two_chunks(m, _):
        k0 = 2 * m
        _fetch(rt, ei, ej, e0 + (k0 + 1) * C, idx_i, idx_j, rows_i, rows_j,
               1, sB1, sB2)
        _drain(rt, idx_i, idx_j, rows_i, rows_j, 0, sA1, sA2)
        compute(k0, 0)

        @pl.when(k0 + 2 < NCHUNK)
        def _prefetch():
            _fetch(rt, ei, ej, e0 + (k0 + 2) * C, idx_i, idx_j, rows_i,
                   rows_j, 0, sA1, sA2)

        _drain(rt, idx_i, idx_j, rows_i, rows_j, 1, sB1, sB2)
        compute(k0 + 1, 1)
        return _

    lax.fori_loop(0, NCHUNK // 2, two_chunks, None)
    _drain(rt, idx_i, idx_j, rows_i, rows_j, 0, sA1, sA2)
    compute(NCHUNK - 1, 0)

    plsc.subcore_barrier()
    sl = pl.ds(sid * SLICE, SLICE)
    pltpu.sync_copy(rho_sh.at[sl], rho_out.at[cid, sl])


def _pass2_body(vr, ei, ej, dxc, dyc, dzc, gc, z1, dudt_out,
                idx_i, idx_j, rows_i, rows_j, dxb, dyb, dzb, gb,
                axb, ayb, azb, shx, shy, shz, sA1, sA2, sB1, sB2):
    cid = lax.axis_index("c")
    sid = lax.axis_index("s")
    wid = sid * NC + cid
    e0 = wid * EW
    pltpu.sync_copy(z1, shx.at[pl.ds(sid * SLICE, SLICE)])
    pltpu.sync_copy(z1, shy.at[pl.ds(sid * SLICE, SLICE)])
    pltpu.sync_copy(z1, shz.at[pl.ds(sid * SLICE, SLICE)])
    plsc.subcore_barrier()
    iota = lax.iota(jnp.int32, L)
    c0 = jnp.full((L,), 0, jnp.int32)
    c1 = jnp.full((L,), 1, jnp.int32)
    c2 = jnp.full((L,), 2, jnp.int32)
    c3 = jnp.full((L,), 3, jnp.int32)

    def compute(k, s):
        rows_i_s = _slot(rows_i, s)
        rows_j_s = _slot(rows_j, s)
        base = pl.multiple_of(e0 + k * C, 8)
        pltpu.sync_copy(dxc.at[pl.ds(base, C)], dxb)
        pltpu.sync_copy(dyc.at[pl.ds(base, C)], dyb)
        pltpu.sync_copy(dzc.at[pl.ds(base, C)], dzb)
        pltpu.sync_copy(gc.at[pl.ds(base, C)], gb)

        def grp(l, _):
            row = l * L + iota
            vix = plsc.load_gather(rows_i_s, [row, c0])
            viy = plsc.load_gather(rows_i_s, [row, c1])
            viz = plsc.load_gather(rows_i_s, [row, c2])
            ri = plsc.load_gather(rows_i_s, [row, c3])
            vjx = plsc.load_gather(rows_j_s, [row, c0])
            vjy = plsc.load_gather(rows_j_s, [row, c1])
            vjz = plsc.load_gather(rows_j_s, [row, c2])
            rj = plsc.load_gather(rows_j_s, [row, c3])
            sl = pl.ds(l * L, L)
            dx = dxb[sl]
            dy = dyb[sl]
            dz = dzb[sl]
            g = gb[sl]
            inv_i = 1.0 / ri
            inv_j = 1.0 / rj
            cc = (inv_i * inv_i + inv_j * inv_j) * g
            num = P_REF * (2.0 * ri * rj - ri - rj)
            p_ij = num / (ri + rj)
            axb[sl] = cc * (-p_ij * dx + ETA_IJ * (vix - vjx))
            ayb[sl] = cc * (-p_ij * dy + ETA_IJ * (viy - vjy))
            azb[sl] = cc * (-p_ij * dz + ETA_IJ * (viz - vjz))
            return _

        lax.fori_loop(0, C // L, grp, None)
        pltpu.sync_copy(axb, shx.at[_slot(idx_i, s)], add=True)
        pltpu.sync_copy(ayb, shy.at[_slot(idx_i, s)], add=True)
        pltpu.sync_copy(azb, shz.at[_slot(idx_i, s)], add=True)

    _fetch(vr, ei, ej, e0, idx_i, idx_j, rows_i, rows_j, 0, sA1, sA2)

    def two_chunks(m, _):
        k0 = 2 * m
        _fetch(vr, ei, ej, e0 + (k0 + 1) * C, idx_i, idx_j, rows_i, rows_j,
               1, sB1, sB2)
        _drain(vr, idx_i, idx_j, rows_i, rows_j, 0, sA1, sA2)
        compute(k0, 0)

        @pl.when(k0 + 2 < NCHUNK)
        def _prefetch():
            _fetch(vr, ei, ej, e0 + (k0 + 2) * C, idx_i, idx_j, rows_i,
                   rows_j, 0, sA1, sA2)

        _drain(vr, idx_i, idx_j, rows_i, rows_j, 1, sB1, sB2)
        compute(k0 + 1, 1)
        return _

    lax.fori_loop(0, NCHUNK // 2, two_chunks, None)
    _drain(vr, idx_i, idx_j, rows_i, rows_j, 0, sA1, sA2)
    compute(NCHUNK - 1, 0)

    plsc.subcore_barrier()
    sl = pl.ds(sid * SLICE, SLICE)
    pltpu.sync_copy(shx.at[sl], dudt_out.at[cid, 0, sl])
    pltpu.sync_copy(shy.at[sl], dudt_out.at[cid, 1, sl])
    pltpu.sync_copy(shz.at[sl], dudt_out.at[cid, 2, sl])


def _sum_rho_body(part_ref, rho_ref, p_ref):
    s = part_ref[0] + part_ref[1]
    rho_ref[...] = s
    p_ref[...] = P_REF * (s - 1.0)


def _sum_dudt_body(part_ref, out_ref):
    out_ref[...] = part_ref[0] + part_ref[1]


@jax.jit
def kernel(abs_pos, vel_hist, edge_index, tag):
    del tag
    f32 = jnp.float32
    r = abs_pos[..., -1].astype(f32)
    rt = jnp.pad(r, ((0, 0), (0, 5)))
    ei = edge_index[0].astype(jnp.int32)
    ej = edge_index[1].astype(jnp.int32)
    z1 = jnp.zeros((SLICE,), f32)

    mesh = plsc.VectorSubcoreMesh(
        core_axis_name="c", subcore_axis_name="s",
        num_cores=NC, num_subcores=NS)
    cparams = pltpu.CompilerParams(
        use_tc_tiling_on_sc=False, needs_layout_passes=False)

    pass1 = pl.kernel(
        _pass1_body,
        out_type=[
            jax.ShapeDtypeStruct((NC, N2), f32),
            jax.ShapeDtypeStruct((E,), f32),
            jax.ShapeDtypeStruct((E,), f32),
            jax.ShapeDtypeStruct((E,), f32),
            jax.ShapeDtypeStruct((E,), f32),
        ],
        mesh=mesh,
        scratch_types=[
            pltpu.VMEM((2 * C,), jnp.int32),
            pltpu.VMEM((2 * C,), jnp.int32),
            pltpu.VMEM((2 * C, 8), f32),
            pltpu.VMEM((2 * C, 8), f32),
            pltpu.VMEM((C,), f32),
            pltpu.VMEM((C,), f32),
            pltpu.VMEM((C,), f32),
            pltpu.VMEM((C,), f32),
            pltpu.VMEM((C,), f32),
            pltpu.VMEM_SHARED((N2,), f32),
            pltpu.SemaphoreType.DMA,
            pltpu.SemaphoreType.DMA,
            pltpu.SemaphoreType.DMA,
            pltpu.SemaphoreType.DMA,
        ],
        compiler_params=cparams,
    )
    rho_part, dxc, dyc, dzc, gc = pass1(rt, ei, ej, z1)

    rho2d, p2d = pl.pallas_call(
        _sum_rho_body,
        out_shape=[
            jax.ShapeDtypeStruct((N2 // 128, 128), f32),
            jax.ShapeDtypeStruct((N2 // 128, 128), f32),
        ],
    )(rho_part.reshape(NC, N2 // 128, 128))
    rho = rho2d.reshape(-1)[:N]
    p = p2d.reshape(-1)[:N]

    vr = jnp.concatenate(
        [vel_hist.astype(f32), rho[:, None], jnp.zeros((N, 4), f32)],
        axis=1)

    pass2 = pl.kernel(
        _pass2_body,
        out_type=jax.ShapeDtypeStruct((NC, 3, N2), f32),
        mesh=mesh,
        scratch_types=[
            pltpu.VMEM((2 * C,), jnp.int32),
            pltpu.VMEM((2 * C,), jnp.int32),
            pltpu.VMEM((2 * C, 8), f32),
            pltpu.VMEM((2 * C, 8), f32),
            pltpu.VMEM((C,), f32),
            pltpu.VMEM((C,), f32),
            pltpu.VMEM((C,), f32),
            pltpu.VMEM((C,), f32),
            pltpu.VMEM((C,), f32),
            pltpu.VMEM((C,), f32),
            pltpu.VMEM((C,), f32),
            pltpu.VMEM_SHARED((N2,), f32),
            pltpu.VMEM_SHARED((N2,), f32),
            pltpu.VMEM_SHARED((N2,), f32),
            pltpu.SemaphoreType.DMA,
            pltpu.SemaphoreType.DMA,
            pltpu.SemaphoreType.DMA,
            pltpu.SemaphoreType.DMA,
        ],
        compiler_params=cparams,
    )
    dudt_part = pass2(vr, ei, ej, dxc, dyc, dzc, gc, z1)

    dudt3 = pl.pallas_call(
        _sum_dudt_body,
        out_shape=jax.ShapeDtypeStruct((3 * N2 // 128, 128), f32),
    )(dudt_part.reshape(NC, 3 * N2 // 128, 128))
    dudt = dudt3.reshape(3, N2)[:, :N].T

    dvdt = jnp.zeros((N, DIM), f32)
    return dudt, dvdt, rho, p


# speedup vs baseline: 186.2910x; 1.0178x over previous
import jax
import jax.numpy as jnp
from jax import lax
from jax.experimental import pallas as pl
from jax.experimental.pallas import tpu as pltpu
from jax.experimental.pallas import tpu_sc as plsc

N = 50000
DIM = 3
E = 1600000
NC = 2
NS = 16
L = 16
NW = NC * NS
EW = E // NW
C = 2000
NCHUNK = EW // C
N2 = 51200
SLICE = N2 // NS

SIGMA = 3.0 / 359.0 / 3.141592653589793
P_REF = 100.0
ETA_IJ = 2.0 * 0.01 * 0.01 / (0.01 + 0.01 + 1e-08)


def _rsqrt(d2):
    i = plsc.bitcast(d2, jnp.int32)
    y = plsc.bitcast(jnp.int32(0x5F3759DF) - (i >> 1), jnp.float32)
    for _ in range(3):
        y = y * (1.5 - 0.5 * d2 * y * y)
    return y


def _slot(ref, s):
    return ref.at[pl.ds(s * C, C)]


def _fetch(table, ei, ej, base, idx_i, idx_j, rows_i, rows_j, s, semi, semj):
    pltpu.sync_copy(ei.at[pl.ds(base, C)], _slot(idx_i, s))
    pltpu.sync_copy(ej.at[pl.ds(base, C)], _slot(idx_j, s))
    pltpu.async_copy(table.at[_slot(idx_i, s)], _slot(rows_i, s), semi)
    pltpu.async_copy(table.at[_slot(idx_j, s)], _slot(rows_j, s), semj)


def _drain(table, idx_i, idx_j, rows_i, rows_j, s, semi, semj):
    pltpu.make_async_copy(table.at[_slot(idx_i, s)], _slot(rows_i, s),
                          semi).wait()
    pltpu.make_async_copy(table.at[_slot(idx_j, s)], _slot(rows_j, s),
                          semj).wait()


def _pass1_body(rt, ei, ej, z1, rho_out, dxc, dyc, dzc, gc,
                idx_i, idx_j, rows_i, rows_j, wb, dxb, dyb, dzb, gb,
                rho_sh, sA1, sA2, sB1, sB2):
    cid = lax.axis_index("c")
    sid = lax.axis_index("s")
    wid = sid * NC + cid
    e0 = wid * EW
    pltpu.sync_copy(z1, rho_sh.at[pl.ds(sid * SLICE, SLICE)])
    plsc.subcore_barrier()
    iota = lax.iota(jnp.int32, L)
    c0 = jnp.full((L,), 0, jnp.int32)
    c1 = jnp.full((L,), 1, jnp.int32)
    c2 = jnp.full((L,), 2, jnp.int32)

    def compute(k, s):
        rows_i_s = _slot(rows_i, s)
        rows_j_s = _slot(rows_j, s)

        def grp(l, _):
            row = l * L + iota
            rix = plsc.load_gather(rows_i_s, [row, c0])
            riy = plsc.load_gather(rows_i_s, [row, c1])
            riz = plsc.load_gather(rows_i_s, [row, c2])
            rjx = plsc.load_gather(rows_j_s, [row, c0])
            rjy = plsc.load_gather(rows_j_s, [row, c1])
            rjz = plsc.load_gather(rows_j_s, [row, c2])
            dx = rix - rjx
            dy = riy - rjy
            dz = riz - rjz
            d2 = dx * dx + dy * dy + dz * dz + 1e-16
            y = _rsqrt(d2)
            d = d2 * y
            q1 = jnp.maximum(0.0, 1.0 - d)
            q2 = jnp.maximum(0.0, 2.0 - d)
            q3 = jnp.maximum(0.0, 3.0 - d)
            q12 = q1 * q1
            q14 = q12 * q12
            q22 = q2 * q2
            q24 = q22 * q22
            q32 = q3 * q3
            q34 = q32 * q32
            w = SIGMA * (q34 * q3 - 6.0 * (q24 * q2) + 15.0 * (q14 * q1))
            gw = (-5.0 * SIGMA) * (q34 - 6.0 * q24 + 15.0 * q14)
            g = gw / (d + 1e-08)
            sl = pl.ds(l * L, L)
            wb[sl] = w
            dxb[sl] = dx
            dyb[sl] = dy
            dzb[sl] = dz
            gb[sl] = g
            return _

        lax.fori_loop(0, C // L, grp, None)
        base = pl.multiple_of(e0 + k * C, 8)
        pltpu.sync_copy(wb, rho_sh.at[_slot(idx_i, s)], add=True)
        pltpu.sync_copy(dxb, dxc.at[pl.ds(base, C)])
        pltpu.sync_copy(dyb, dyc.at[pl.ds(base, C)])
        pltpu.sync_copy(dzb, dzc.at[pl.ds(base, C)])
        pltpu.sync_copy(gb, gc.at[pl.ds(base, C)])

    _fetch(rt, ei, ej, e0, idx_i, idx_j, rows_i, rows_j, 0, sA1, sA2)

    def two_chunks(m, _):
        k0 = 2 * m
        _fetch(rt, ei, ej, e0 + (k0 + 1) * C, idx_i, idx_j, rows_i, rows_j,
               1, sB1, sB2)
        _drain(rt, idx_i, idx_j, rows_i, rows_j, 0, sA1, sA2)
        compute(k0, 0)

        @pl.when(k0 + 2 < NCHUNK)
        def _prefetch():
            _fetch(rt, ei, ej, e0 + (k0 + 2) * C, idx_i, idx_j, rows_i,
                   rows_j, 0, sA1, sA2)

        _drain(rt, idx_i, idx_j, rows_i, rows_j, 1, sB1, sB2)
        compute(k0 + 1, 1)
        return _

    lax.fori_loop(0, NCHUNK // 2, two_chunks, None)
    _drain(rt, idx_i, idx_j, rows_i, rows_j, 0, sA1, sA2)
    compute(NCHUNK - 1, 0)

    plsc.subcore_barrier()
    sl = pl.ds(sid * SLICE, SLICE)
    pltpu.sync_copy(rho_sh.at[sl], rho_out.at[cid, sl])


def _pass2_body(vp, ei, ej, rho_part, dxc, dyc, dzc, gc, z1,
                dudt_out, vr_t, rho_o, p_o,
                idx_i, idx_j, rows_i, rows_j, dxb, dyb, dzb, gb,
                axb, ayb, azb, vbuf, vrbuf, shx, shy, shz,
                sA1, sA2, sB1, sB2):
    cid = lax.axis_index("c")
    sid = lax.axis_index("s")
    wid = sid * NC + cid
    e0 = wid * EW
    pltpu.sync_copy(z1, shx.at[pl.ds(sid * SLICE, SLICE)])
    pltpu.sync_copy(z1, shy.at[pl.ds(sid * SLICE, SLICE)])
    pltpu.sync_copy(z1, shz.at[pl.ds(sid * SLICE, SLICE)])
    iota = lax.iota(jnp.int32, L)
    c0 = jnp.full((L,), 0, jnp.int32)
    c1 = jnp.full((L,), 1, jnp.int32)
    c2 = jnp.full((L,), 2, jnp.int32)
    c3 = jnp.full((L,), 3, jnp.int32)

    HS = SLICE // 2
    for t in range(2):
        a = pl.multiple_of(sid * SLICE + t * HS, 8)
        pltpu.sync_copy(vp.at[pl.ds(a, HS)], vbuf)
        pltpu.sync_copy(rho_part.at[0, pl.ds(a, HS)], dxb.at[pl.ds(0, HS)])
        pltpu.sync_copy(rho_part.at[1, pl.ds(a, HS)], dyb.at[pl.ds(0, HS)])

        def vrow(l, _):
            row = l * L + iota
            sl = pl.ds(l * L, L)
            rho16 = dxb[sl] + dyb[sl]
            gb[sl] = rho16
            dzb[sl] = P_REF * (rho16 - 1.0)
            vx = plsc.load_gather(vbuf, [row, c0])
            vy = plsc.load_gather(vbuf, [row, c1])
            vz = plsc.load_gather(vbuf, [row, c2])
            plsc.store_scatter(vrbuf, [row, c0], vx)
            plsc.store_scatter(vrbuf, [row, c1], vy)
            plsc.store_scatter(vrbuf, [row, c2], vz)
            plsc.store_scatter(vrbuf, [row, c3], rho16)
            return _

        lax.fori_loop(0, HS // L, vrow, None)
        pltpu.sync_copy(vrbuf, vr_t.at[cid, pl.ds(a, HS)])

        @pl.when(cid == 0)
        def _emit():
            pltpu.sync_copy(gb.at[pl.ds(0, HS)], rho_o.at[pl.ds(a, HS)])
            pltpu.sync_copy(dzb.at[pl.ds(0, HS)], p_o.at[pl.ds(a, HS)])

    plsc.subcore_barrier()
    vr = vr_t.at[cid]

    def compute(k, s):
        rows_i_s = _slot(rows_i, s)
        rows_j_s = _slot(rows_j, s)
        base = pl.multiple_of(e0 + k * C, 8)
        pltpu.sync_copy(dxc.at[pl.ds(base, C)], dxb)
        pltpu.sync_copy(dyc.at[pl.ds(base, C)], dyb)
        pltpu.sync_copy(dzc.at[pl.ds(base, C)], dzb)
        pltpu.sync_copy(gc.at[pl.ds(base, C)], gb)

        def grp(l, _):
            row = l * L + iota
            vix = plsc.load_gather(rows_i_s, [row, c0])
            viy = plsc.load_gather(rows_i_s, [row, c1])
            viz = plsc.load_gather(rows_i_s, [row, c2])
            ri = plsc.load_gather(rows_i_s, [row, c3])
            vjx = plsc.load_gather(rows_j_s, [row, c0])
            vjy = plsc.load_gather(rows_j_s, [row, c1])
            vjz = plsc.load_gather(rows_j_s, [row, c2])
            rj = plsc.load_gather(rows_j_s, [row, c3])
            sl = pl.ds(l * L, L)
            dx = dxb[sl]
            dy = dyb[sl]
            dz = dzb[sl]
            g = gb[sl]
            inv_i = 1.0 / ri
            inv_j = 1.0 / rj
            cc = (inv_i * inv_i + inv_j * inv_j) * g
            num = P_REF * (2.0 * ri * rj - ri - rj)
            p_ij = num / (ri + rj)
            axb[sl] = cc * (-p_ij * dx + ETA_IJ * (vix - vjx))
            ayb[sl] = cc * (-p_ij * dy + ETA_IJ * (viy - vjy))
            azb[sl] = cc * (-p_ij * dz + ETA_IJ * (viz - vjz))
            return _

        lax.fori_loop(0, C // L, grp, None)
        pltpu.sync_copy(axb, shx.at[_slot(idx_i, s)], add=True)
        pltpu.sync_copy(ayb, shy.at[_slot(idx_i, s)], add=True)
        pltpu.sync_copy(azb, shz.at[_slot(idx_i, s)], add=True)

    _fetch(vr, ei, ej, e0, idx_i, idx_j, rows_i, rows_j, 0, sA1, sA2)

    def two_chunks(m, _):
        k0 = 2 * m
        _fetch(vr, ei, ej, e0 + (k0 + 1) * C, idx_i, idx_j, rows_i, rows_j,
               1, sB1, sB2)
        _drain(vr, idx_i, idx_j, rows_i, rows_j, 0, sA1, sA2)
        compute(k0, 0)

        @pl.when(k0 + 2 < NCHUNK)
        def _prefetch():
            _fetch(vr, ei, ej, e0 + (k0 + 2) * C, idx_i, idx_j, rows_i,
                   rows_j, 0, sA1, sA2)

        _drain(vr, idx_i, idx_j, rows_i, rows_j, 1, sB1, sB2)
        compute(k0 + 1, 1)
        return _

    lax.fori_loop(0, NCHUNK // 2, two_chunks, None)
    _drain(vr, idx_i, idx_j, rows_i, rows_j, 0, sA1, sA2)
    compute(NCHUNK - 1, 0)

    plsc.subcore_barrier()
    sl = pl.ds(sid * SLICE, SLICE)
    pltpu.sync_copy(shx.at[sl], dudt_out.at[cid, 0, sl])
    pltpu.sync_copy(shy.at[sl], dudt_out.at[cid, 1, sl])
    pltpu.sync_copy(shz.at[sl], dudt_out.at[cid, 2, sl])


def _sum_dudt_body(part_ref, out_ref):
    out_ref[...] = part_ref[0] + part_ref[1]


@jax.jit
def kernel(abs_pos, vel_hist, edge_index, tag):
    del tag
    f32 = jnp.float32
    r = abs_pos[..., -1].astype(f32)
    rt = jnp.pad(r, ((0, 0), (0, 5)))
    ei = edge_index[0].astype(jnp.int32)
    ej = edge_index[1].astype(jnp.int32)
    z1 = jnp.zeros((SLICE,), f32)

    mesh = plsc.VectorSubcoreMesh(
        core_axis_name="c", subcore_axis_name="s",
        num_cores=NC, num_subcores=NS)
    cparams = pltpu.CompilerParams(
        use_tc_tiling_on_sc=False, needs_layout_passes=False)

    pass1 = pl.kernel(
        _pass1_body,
        out_type=[
            jax.ShapeDtypeStruct((NC, N2), f32),
            jax.ShapeDtypeStruct((E,), f32),
            jax.ShapeDtypeStruct((E,), f32),
            jax.ShapeDtypeStruct((E,), f32),
            jax.ShapeDtypeStruct((E,), f32),
        ],
        mesh=mesh,
        scratch_types=[
            pltpu.VMEM((2 * C,), jnp.int32),
            pltpu.VMEM((2 * C,), jnp.int32),
            pltpu.VMEM((2 * C, 8), f32),
            pltpu.VMEM((2 * C, 8), f32),
            pltpu.VMEM((C,), f32),
            pltpu.VMEM((C,), f32),
            pltpu.VMEM((C,), f32),
            pltpu.VMEM((C,), f32),
            pltpu.VMEM((C,), f32),
            pltpu.VMEM_SHARED((N2,), f32),
            pltpu.SemaphoreType.DMA,
            pltpu.SemaphoreType.DMA,
            pltpu.SemaphoreType.DMA,
            pltpu.SemaphoreType.DMA,
        ],
        compiler_params=cparams,
    )
    rho_part, dxc, dyc, dzc, gc = pass1(rt, ei, ej, z1)

    vp = jnp.pad(vel_hist.astype(f32), ((0, N2 - N), (0, 0)))

    pass2 = pl.kernel(
        _pass2_body,
        out_type=[
            jax.ShapeDtypeStruct((NC, 3, N2), f32),
            jax.ShapeDtypeStruct((NC, N2, 8), f32),
            jax.ShapeDtypeStruct((N2,), f32),
            jax.ShapeDtypeStruct((N2,), f32),
        ],
        mesh=mesh,
        scratch_types=[
            pltpu.VMEM((2 * C,), jnp.int32),
            pltpu.VMEM((2 * C,), jnp.int32),
            pltpu.VMEM((2 * C, 8), f32),
            pltpu.VMEM((2 * C, 8), f32),
            pltpu.VMEM((C,), f32),
            pltpu.VMEM((C,), f32),
            pltpu.VMEM((C,), f32),
            pltpu.VMEM((C,), f32),
            pltpu.VMEM((C,), f32),
            pltpu.VMEM((C,), f32),
            pltpu.VMEM((C,), f32),
            pltpu.VMEM((SLICE // 2, 3), f32),
            pltpu.VMEM((SLICE // 2, 8), f32),
            pltpu.VMEM_SHARED((N2,), f32),
            pltpu.VMEM_SHARED((N2,), f32),
            pltpu.VMEM_SHARED((N2,), f32),
            pltpu.SemaphoreType.DMA,
            pltpu.SemaphoreType.DMA,
            pltpu.SemaphoreType.DMA,
            pltpu.SemaphoreType.DMA,
        ],
        compiler_params=cparams,
    )
    dudt_part, _vr_t, rho_o, p_o = pass2(
        vp, ei, ej, rho_part, dxc, dyc, dzc, gc, z1)
    rho = rho_o[:N]
    p = p_o[:N]

    dudt3 = pl.pallas_call(
        _sum_dudt_body,
        out_shape=jax.ShapeDtypeStruct((3 * N2 // 128, 128), f32),
    )(dudt_part.reshape(NC, 3 * N2 // 128, 128))
    dudt = dudt3.reshape(3, N2)[:, :N].T

    dvdt = jnp.zeros((N, DIM), f32)
    return dudt, dvdt, rho, p


# speedup vs baseline: 192.0701x; 1.0310x over previous
import jax
import jax.numpy as jnp
from jax import lax
from jax.experimental import pallas as pl
from jax.experimental.pallas import tpu as pltpu
from jax.experimental.pallas import tpu_sc as plsc

N = 50000
DIM = 3
E = 1600000
NC = 2
NS = 16
L = 16
NW = NC * NS
EW = E // NW
C = 2000
NCHUNK = EW // C
N2 = 51200
SLICE = N2 // NS
NA = 50048
SLA = NA // NS

SIGMA = 3.0 / 359.0 / 3.141592653589793
P_REF = 100.0
ETA_IJ = 2.0 * 0.01 * 0.01 / (0.01 + 0.01 + 1e-08)


def _rsqrt(d2):
    i = plsc.bitcast(d2, jnp.int32)
    y = plsc.bitcast(jnp.int32(0x5F3759DF) - (i >> 1), jnp.float32)
    for _ in range(3):
        y = y * (1.5 - 0.5 * d2 * y * y)
    return y


def _slot(ref, s):
    return ref.at[pl.ds(s * C, C)]


def _fetch(table, ei, ej, base, idx_i, idx_j, rows_i, rows_j, s, semi, semj):
    pltpu.sync_copy(ei.at[pl.ds(base, C)], _slot(idx_i, s))
    pltpu.sync_copy(ej.at[pl.ds(base, C)], _slot(idx_j, s))
    pltpu.async_copy(table.at[_slot(idx_i, s)], _slot(rows_i, s), semi)
    pltpu.async_copy(table.at[_slot(idx_j, s)], _slot(rows_j, s), semj)


def _drain(table, idx_i, idx_j, rows_i, rows_j, s, semi, semj):
    pltpu.make_async_copy(table.at[_slot(idx_i, s)], _slot(rows_i, s),
                          semi).wait()
    pltpu.make_async_copy(table.at[_slot(idx_j, s)], _slot(rows_j, s),
                          semj).wait()


def _pass1_body(rt, ei, ej, z1, rho_out, dxc, dyc, dzc, gc,
                idx_i, idx_j, sidx, rows_i, rows_j, wb, dxb, dyb, dzb, gb,
                rho_sh, sA1, sA2, sB1, sB2, sS0, sS1):
    cid = lax.axis_index("c")
    sid = lax.axis_index("s")
    wid = sid * NC + cid
    e0 = wid * EW
    pltpu.sync_copy(z1, rho_sh.at[pl.ds(sid * SLICE, SLICE)])
    plsc.subcore_barrier()
    iota = lax.iota(jnp.int32, L)
    c0 = jnp.full((L,), 0, jnp.int32)
    c1 = jnp.full((L,), 1, jnp.int32)
    c2 = jnp.full((L,), 2, jnp.int32)

    def compute(k, s, first):
        rows_i_s = _slot(rows_i, s)
        rows_j_s = _slot(rows_j, s)
        ssem = sS0 if s == 0 else sS1
        sidx_s = _slot(sidx, s)
        wb_s = _slot(wb, s)

        def _drain_scat():
            pltpu.make_async_copy(wb_s, rho_sh.at[sidx_s], ssem).wait()

        if first:
            pass
        elif isinstance(k, int):
            _drain_scat()
        else:
            pl.when(k >= 2)(_drain_scat)
        pltpu.sync_copy(ei.at[pl.ds(pl.multiple_of(e0 + k * C, 8), C)],
                        sidx_s)

        def grp(l, _):
            row = l * L + iota
            rix = plsc.load_gather(rows_i_s, [row, c0])
            riy = plsc.load_gather(rows_i_s, [row, c1])
            riz = plsc.load_gather(rows_i_s, [row, c2])
            rjx = plsc.load_gather(rows_j_s, [row, c0])
            rjy = plsc.load_gather(rows_j_s, [row, c1])
            rjz = plsc.load_gather(rows_j_s, [row, c2])
            dx = rix - rjx
            dy = riy - rjy
            dz = riz - rjz
            d2 = dx * dx + dy * dy + dz * dz + 1e-16
            y = _rsqrt(d2)
            d = d2 * y
            q1 = jnp.maximum(0.0, 1.0 - d)
            q2 = jnp.maximum(0.0, 2.0 - d)
            q3 = jnp.maximum(0.0, 3.0 - d)
            q12 = q1 * q1
            q14 = q12 * q12
            q22 = q2 * q2
            q24 = q22 * q22
            q32 = q3 * q3
            q34 = q32 * q32
            w = SIGMA * (q34 * q3 - 6.0 * (q24 * q2) + 15.0 * (q14 * q1))
            gw = (-5.0 * SIGMA) * (q34 - 6.0 * q24 + 15.0 * q14)
            g = gw / (d + 1e-08)
            sl = pl.ds(l * L, L)
            wb_s[sl] = w
            dxb[sl] = dx
            dyb[sl] = dy
            dzb[sl] = dz
            gb[sl] = g
            return _

        lax.fori_loop(0, C // L, grp, None)
        base = pl.multiple_of(e0 + k * C, 8)
        pltpu.async_copy(wb_s, rho_sh.at[sidx_s], ssem, add=True)
        pltpu.sync_copy(dxb, dxc.at[pl.ds(base, C)])
        pltpu.sync_copy(dyb, dyc.at[pl.ds(base, C)])
        pltpu.sync_copy(dzb, dzc.at[pl.ds(base, C)])
        pltpu.sync_copy(gb, gc.at[pl.ds(base, C)])

    _fetch(rt, ei, ej, e0, idx_i, idx_j, rows_i, rows_j, 0, sA1, sA2)

    def two_chunks(m, _):
        k0 = 2 * m
        _fetch(rt, ei, ej, e0 + (k0 + 1) * C, idx_i, idx_j, rows_i, rows_j,
               1, sB1, sB2)
        _drain(rt, idx_i, idx_j, rows_i, rows_j, 0, sA1, sA2)
        compute(k0, 0, False)

        @pl.when(k0 + 2 < NCHUNK)
        def _prefetch():
            _fetch(rt, ei, ej, e0 + (k0 + 2) * C, idx_i, idx_j, rows_i,
                   rows_j, 0, sA1, sA2)

        _drain(rt, idx_i, idx_j, rows_i, rows_j, 1, sB1, sB2)
        compute(k0 + 1, 1, False)
        return _

    lax.fori_loop(0, NCHUNK // 2, two_chunks, None)
    _drain(rt, idx_i, idx_j, rows_i, rows_j, 0, sA1, sA2)
    compute(NCHUNK - 1, 0, False)
    pltpu.make_async_copy(_slot(wb, 0), rho_sh.at[_slot(sidx, 0)], sS0).wait()
    pltpu.make_async_copy(_slot(wb, 1), rho_sh.at[_slot(sidx, 1)], sS1).wait()

    plsc.subcore_barrier()
    sl = pl.ds(sid * SLICE, SLICE)
    pltpu.sync_copy(rho_sh.at[sl], rho_out.at[cid, sl])


def _pass2_body(vp, ei, ej, rho_part, dxc, dyc, dzc, gc, z1,
                dudt_out, vr_t, rho_o, p_o,
                idx_i, idx_j, sidx, rows_i, rows_j, dxb, dyb, dzb, gb,
                axb, ayb, azb, vbuf, vrbuf, shx, shy, shz,
                sA1, sA2, sB1, sB2, sS0, sS1):
    cid = lax.axis_index("c")
    sid = lax.axis_index("s")
    wid = sid * NC + cid
    e0 = wid * EW
    zs = pl.ds(sid * SLA, SLA)
    pltpu.sync_copy(z1.at[pl.ds(0, SLA)], shx.at[zs])
    pltpu.sync_copy(z1.at[pl.ds(0, SLA)], shy.at[zs])
    pltpu.sync_copy(z1.at[pl.ds(0, SLA)], shz.at[zs])
    iota = lax.iota(jnp.int32, L)
    c0 = jnp.full((L,), 0, jnp.int32)
    c1 = jnp.full((L,), 1, jnp.int32)
    c2 = jnp.full((L,), 2, jnp.int32)
    c3 = jnp.full((L,), 3, jnp.int32)

    HS = SLICE // 2
    for t in range(2):
        a = pl.multiple_of(sid * SLICE + t * HS, 8)
        pltpu.sync_copy(vp.at[pl.ds(a, HS)], vbuf)
        pltpu.sync_copy(rho_part.at[0, pl.ds(a, HS)], dxb.at[pl.ds(0, HS)])
        pltpu.sync_copy(rho_part.at[1, pl.ds(a, HS)], dyb.at[pl.ds(0, HS)])

        def vrow(l, _):
            row = l * L + iota
            sl = pl.ds(l * L, L)
            rho16 = dxb[sl] + dyb[sl]
            gb[sl] = rho16
            dzb[sl] = P_REF * (rho16 - 1.0)
            vx = plsc.load_gather(vbuf, [row, c0])
            vy = plsc.load_gather(vbuf, [row, c1])
            vz = plsc.load_gather(vbuf, [row, c2])
            plsc.store_scatter(vrbuf, [row, c0], vx)
            plsc.store_scatter(vrbuf, [row, c1], vy)
            plsc.store_scatter(vrbuf, [row, c2], vz)
            plsc.store_scatter(vrbuf, [row, c3], rho16)
            return _

        lax.fori_loop(0, HS // L, vrow, None)
        pltpu.sync_copy(vrbuf, vr_t.at[cid, pl.ds(a, HS)])

        @pl.when(cid == 0)
        def _emit():
            pltpu.sync_copy(gb.at[pl.ds(0, HS)], rho_o.at[pl.ds(a, HS)])
            pltpu.sync_copy(dzb.at[pl.ds(0, HS)], p_o.at[pl.ds(a, HS)])

    plsc.subcore_barrier()
    vr = vr_t.at[cid]

    def compute(k, s):
        rows_i_s = _slot(rows_i, s)
        rows_j_s = _slot(rows_j, s)
        ssem = sS0 if s == 0 else sS1
        sidx_s = _slot(sidx, s)
        axb_s = _slot(axb, s)
        ayb_s = _slot(ayb, s)
        azb_s = _slot(azb, s)

        def _drain_scat():
            pltpu.make_async_copy(axb_s, shx.at[sidx_s], ssem).wait()
            pltpu.make_async_copy(ayb_s, shy.at[sidx_s], ssem).wait()
            pltpu.make_async_copy(azb_s, shz.at[sidx_s], ssem).wait()

        if isinstance(k, int):
            _drain_scat()
        else:
            pl.when(k >= 2)(_drain_scat)
        base = pl.multiple_of(e0 + k * C, 8)
        pltpu.sync_copy(ei.at[pl.ds(base, C)], sidx_s)
        pltpu.sync_copy(dxc.at[pl.ds(base, C)], dxb)
        pltpu.sync_copy(dyc.at[pl.ds(base, C)], dyb)
        pltpu.sync_copy(dzc.at[pl.ds(base, C)], dzb)
        pltpu.sync_copy(gc.at[pl.ds(base, C)], gb)

        def grp(l, _):
            row = l * L + iota
            vix = plsc.load_gather(rows_i_s, [row, c0])
            viy = plsc.load_gather(rows_i_s, [row, c1])
            viz = plsc.load_gather(rows_i_s, [row, c2])
            ri = plsc.load_gather(rows_i_s, [row, c3])
            vjx = plsc.load_gather(rows_j_s, [row, c0])
            vjy = plsc.load_gather(rows_j_s, [row, c1])
            vjz = plsc.load_gather(rows_j_s, [row, c2])
            rj = plsc.load_gather(rows_j_s, [row, c3])
            sl = pl.ds(l * L, L)
            dx = dxb[sl]
            dy = dyb[sl]
            dz = dzb[sl]
            g = gb[sl]
            inv_i = 1.0 / ri
            inv_j = 1.0 / rj
            cc = (inv_i * inv_i + inv_j * inv_j) * g
            num = P_REF * (2.0 * ri * rj - ri - rj)
            p_ij = num / (ri + rj)
            axb_s[sl] = cc * (-p_ij * dx + ETA_IJ * (vix - vjx))
            ayb_s[sl] = cc * (-p_ij * dy + ETA_IJ * (viy - vjy))
            azb_s[sl] = cc * (-p_ij * dz + ETA_IJ * (viz - vjz))
            return _

        lax.fori_loop(0, C // L, grp, None)
        pltpu.async_copy(axb_s, shx.at[sidx_s], ssem, add=True)
        pltpu.async_copy(ayb_s, shy.at[sidx_s], ssem, add=True)
        pltpu.async_copy(azb_s, shz.at[sidx_s], ssem, add=True)

    _fetch(vr, ei, ej, e0, idx_i, idx_j, rows_i, rows_j, 0, sA1, sA2)

    def two_chunks(m, _):
        k0 = 2 * m
        _fetch(vr, ei, ej, e0 + (k0 + 1) * C, idx_i, idx_j, rows_i, rows_j,
               1, sB1, sB2)
        _drain(vr, idx_i, idx_j, rows_i, rows_j, 0, sA1, sA2)
        compute(k0, 0)

        @pl.when(k0 + 2 < NCHUNK)
        def _prefetch():
            _fetch(vr, ei, ej, e0 + (k0 + 2) * C, idx_i, idx_j, rows_i,
                   rows_j, 0, sA1, sA2)

        _drain(vr, idx_i, idx_j, rows_i, rows_j, 1, sB1, sB2)
        compute(k0 + 1, 1)
        return _

    lax.fori_loop(0, NCHUNK // 2, two_chunks, None)
    _drain(vr, idx_i, idx_j, rows_i, rows_j, 0, sA1, sA2)
    compute(NCHUNK - 1, 0)
    for s, ssem in ((0, sS0), (1, sS1)):
        pltpu.make_async_copy(_slot(axb, s), shx.at[_slot(sidx, s)], ssem).wait()
        pltpu.make_async_copy(_slot(ayb, s), shy.at[_slot(sidx, s)], ssem).wait()
        pltpu.make_async_copy(_slot(azb, s), shz.at[_slot(sidx, s)], ssem).wait()

    plsc.subcore_barrier()
    sl = pl.ds(sid * SLA, SLA)
    pltpu.sync_copy(shx.at[sl], dudt_out.at[cid, 0, sl])
    pltpu.sync_copy(shy.at[sl], dudt_out.at[cid, 1, sl])
    pltpu.sync_copy(shz.at[sl], dudt_out.at[cid, 2, sl])


def _sum_dudt_body(part_ref, out_ref):
    out_ref[...] = part_ref[0] + part_ref[1]


@jax.jit
def kernel(abs_pos, vel_hist, edge_index, tag):
    del tag
    f32 = jnp.float32
    r = abs_pos[..., -1].astype(f32)
    rt = jnp.pad(r, ((0, 0), (0, 5)))
    ei = edge_index[0].astype(jnp.int32)
    ej = edge_index[1].astype(jnp.int32)
    z1 = jnp.zeros((SLICE,), f32)

    mesh = plsc.VectorSubcoreMesh(
        core_axis_name="c", subcore_axis_name="s",
        num_cores=NC, num_subcores=NS)
    cparams = pltpu.CompilerParams(
        use_tc_tiling_on_sc=False, needs_layout_passes=False)

    pass1 = pl.kernel(
        _pass1_body,
        out_type=[
            jax.ShapeDtypeStruct((NC, N2), f32),
            jax.ShapeDtypeStruct((E,), f32),
            jax.ShapeDtypeStruct((E,), f32),
            jax.ShapeDtypeStruct((E,), f32),
            jax.ShapeDtypeStruct((E,), f32),
        ],
        mesh=mesh,
        scratch_types=[
            pltpu.VMEM((2 * C,), jnp.int32),
            pltpu.VMEM((2 * C,), jnp.int32),
            pltpu.VMEM((2 * C,), jnp.int32),
            pltpu.VMEM((2 * C, 8), f32),
            pltpu.VMEM((2 * C, 8), f32),
            pltpu.VMEM((2 * C,), f32),
            pltpu.VMEM((C,), f32),
            pltpu.VMEM((C,), f32),
            pltpu.VMEM((C,), f32),
            pltpu.VMEM((C,), f32),
            pltpu.VMEM_SHARED((N2,), f32),
            pltpu.SemaphoreType.DMA,
            pltpu.SemaphoreType.DMA,
            pltpu.SemaphoreType.DMA,
            pltpu.SemaphoreType.DMA,
            pltpu.SemaphoreType.DMA,
            pltpu.SemaphoreType.DMA,
        ],
        compiler_params=cparams,
    )
    rho_part, dxc, dyc, dzc, gc = pass1(rt, ei, ej, z1)

    vp = jnp.pad(vel_hist.astype(f32), ((0, N2 - N), (0, 0)))

    pass2 = pl.kernel(
        _pass2_body,
        out_type=[
            jax.ShapeDtypeStruct((NC, 3, NA), f32),
            jax.ShapeDtypeStruct((NC, N2, 8), f32),
            jax.ShapeDtypeStruct((N2,), f32),
            jax.ShapeDtypeStruct((N2,), f32),
        ],
        mesh=mesh,
        scratch_types=[
            pltpu.VMEM((2 * C,), jnp.int32),
            pltpu.VMEM((2 * C,), jnp.int32),
            pltpu.VMEM((2 * C,), jnp.int32),
            pltpu.VMEM((2 * C, 8), f32),
            pltpu.VMEM((2 * C, 8), f32),
            pltpu.VMEM((C,), f32),
            pltpu.VMEM((C,), f32),
            pltpu.VMEM((C,), f32),
            pltpu.VMEM((C,), f32),
            pltpu.VMEM((2 * C,), f32),
            pltpu.VMEM((2 * C,), f32),
            pltpu.VMEM((2 * C,), f32),
            pltpu.VMEM((SLICE // 2, 3), f32),
            pltpu.VMEM((SLICE // 2, 8), f32),
            pltpu.VMEM_SHARED((NA,), f32),
            pltpu.VMEM_SHARED((NA,), f32),
            pltpu.VMEM_SHARED((NA,), f32),
            pltpu.SemaphoreType.DMA,
            pltpu.SemaphoreType.DMA,
            pltpu.SemaphoreType.DMA,
            pltpu.SemaphoreType.DMA,
            pltpu.SemaphoreType.DMA,
            pltpu.SemaphoreType.DMA,
        ],
        compiler_params=cparams,
    )
    dudt_part, _vr_t, rho_o, p_o = pass2(
        vp, ei, ej, rho_part, dxc, dyc, dzc, gc, z1)
    rho = rho_o[:N]
    p = p_o[:N]

    dudt3 = pl.pallas_call(
        _sum_dudt_body,
        out_shape=jax.ShapeDtypeStruct((3 * NA // 128, 128), f32),
    )(dudt_part.reshape(NC, 3 * NA // 128, 128))
    dudt = dudt3.reshape(3, NA)[:, :N].T

    dvdt = jnp.zeros((N, DIM), f32)
    return dudt, dvdt, rho, p
